# baseline (speedup 1.0000x reference)
import functools

import jax
import jax.numpy as jnp
from jax import lax
from jax.experimental import pallas as pl
from jax.experimental.pallas import tpu as pltpu
from jax.experimental.pallas import tpu_sc as plsc

_NUM_CORES = 2
_NUM_SUBCORES = 16
_LANES = 16
_NW = _NUM_CORES * _NUM_SUBCORES

_N_ENT = 100000
_N_REL = 1000
_D = 64

_ENT_TOT = _N_ENT * _D
_REL_TOT = _N_REL * _D
_ENT_PER_W = _ENT_TOT // _NW
_REL_PER_W = _REL_TOT // _NW

_CH = 50_000
_N_CHUNKS = _ENT_PER_W // _CH
_U = 5


def _sc_add_body(e0, e1, r0, r1, out_e, out_r, ba, bb, ra, rb):
    wid = lax.axis_index("s") * _NUM_CORES + lax.axis_index("c")

    ent_base = wid * _ENT_PER_W
    for c in range(_N_CHUNKS):
        base = ent_base + c * _CH

        pltpu.sync_copy(e0.at[pl.ds(base, _CH)], ba)
        pltpu.sync_copy(e1.at[pl.ds(base, _CH)], bb)

        def add_step(i, _):
            for j in range(_U):
                off = i * (_LANES * _U) + j * _LANES
                ba[pl.ds(off, _LANES)] = (
                    ba[pl.ds(off, _LANES)] + bb[pl.ds(off, _LANES)]
                )
            return 0

        lax.fori_loop(0, _CH // (_LANES * _U), add_step, 0)
        pltpu.sync_copy(ba, out_e.at[pl.ds(base, _CH)])

    rel_base = wid * _REL_PER_W
    pltpu.sync_copy(r0.at[pl.ds(rel_base, _REL_PER_W)], ra)
    pltpu.sync_copy(r1.at[pl.ds(rel_base, _REL_PER_W)], rb)

    def rel_step(i, _):
        for j in range(_U):
            off = i * (_LANES * _U) + j * _LANES
            ra[pl.ds(off, _LANES)] = (
                ra[pl.ds(off, _LANES)] + rb[pl.ds(off, _LANES)]
            )
        return 0

    lax.fori_loop(0, _REL_PER_W // (_LANES * _U), rel_step, 0)
    pltpu.sync_copy(ra, out_r.at[pl.ds(rel_base, _REL_PER_W)])


_sc_add = pl.kernel(
    _sc_add_body,
    out_type=(
        jax.ShapeDtypeStruct((_ENT_TOT,), jnp.float32),
        jax.ShapeDtypeStruct((_REL_TOT,), jnp.float32),
    ),
    mesh=plsc.VectorSubcoreMesh(
        core_axis_name="c",
        subcore_axis_name="s",
        num_cores=_NUM_CORES,
        num_subcores=_NUM_SUBCORES,
    ),
    scratch_types=[
        pltpu.VMEM((_CH,), jnp.float32),
        pltpu.VMEM((_CH,), jnp.float32),
        pltpu.VMEM((_REL_PER_W,), jnp.float32),
        pltpu.VMEM((_REL_PER_W,), jnp.float32),
    ],
)


def kernel(inputs, ent_embeds_0, rel_embeds_0, ent_embeds_1, rel_embeds_1):
    del inputs
    out_ent, out_rel = _sc_add(
        ent_embeds_0.reshape(_ENT_TOT),
        ent_embeds_1.reshape(_ENT_TOT),
        rel_embeds_0.reshape(_REL_TOT),
        rel_embeds_1.reshape(_REL_TOT),
    )
    return (out_ent.reshape(_N_ENT, _D), out_rel.reshape(_N_REL, _D))


# speedup vs baseline: 1.2998x; 1.2998x over previous
import jax
import jax.numpy as jnp
from jax import lax
from jax.experimental import pallas as pl
from jax.experimental.pallas import tpu as pltpu
from jax.experimental.pallas import tpu_sc as plsc

_NUM_CORES = 2
_NUM_SUBCORES = 16
_LANES = 16
_NW = _NUM_CORES * _NUM_SUBCORES

_N_ENT = 100000
_N_REL = 1000
_D = 64

_ROWS_PW = 3128
_CH = 136
_NCH = _ROWS_PW // _CH
_ENT_LAST = _N_ENT - _ROWS_PW

_RCH = 128
_REL_WORKERS = 8
_REL_LAST = _N_REL - _RCH


def _add_chunk(a, b, rows):
    def row_step(i, _):
        for j in range(_D // _LANES):
            sl = pl.ds(j * _LANES, _LANES)
            a[i, sl] = a[i, sl] + b[i, sl]
        return 0

    lax.fori_loop(0, rows, row_step, 0)


def _sc_add_body(e0, e1, r0, r1, out_e, out_r,
                 a0, a1, b0, b1, sl0, sl1, ss0, ss1):
    wid = lax.axis_index("s") * _NUM_CORES + lax.axis_index("c")
    base = pl.multiple_of(jnp.minimum(wid * _ROWS_PW, _ENT_LAST), 8)

    abuf = (a0, a1)
    bbuf = (b0, b1)
    lsem = (sl0, sl1)
    ssem = (ss0, ss1)
    load_h = [None, None]
    store_h = [None, None]

    def issue_loads(slot, k):
        rows = pl.ds(pl.multiple_of(base + k * _CH, 8), _CH)
        ha = pltpu.async_copy(e0.at[rows], abuf[slot], lsem[slot])
        hb = pltpu.async_copy(e1.at[rows], bbuf[slot], lsem[slot])
        return (ha, hb)

    load_h[0] = issue_loads(0, 0)

    for k in range(_NCH):
        s = k & 1
        ns = (k + 1) & 1
        if k + 1 < _NCH:
            if store_h[ns] is not None:
                store_h[ns].wait()
                store_h[ns] = None
            load_h[ns] = issue_loads(ns, k + 1)
        for h in load_h[s]:
            h.wait()
        _add_chunk(abuf[s], bbuf[s], _CH)
        dst = pl.ds(pl.multiple_of(base + k * _CH, 8), _CH)
        store_h[s] = pltpu.async_copy(abuf[s], out_e.at[dst], ssem[s])

    for h in store_h:
        if h is not None:
            h.wait()

    @pl.when(wid < _REL_WORKERS)
    def _():
        rbase = pl.multiple_of(jnp.minimum(wid * _RCH, _REL_LAST), 8)
        rows = pl.ds(rbase, _RCH)
        va = a0.at[pl.ds(0, _RCH)]
        vb = b0.at[pl.ds(0, _RCH)]
        pltpu.sync_copy(r0.at[rows], va)
        pltpu.sync_copy(r1.at[rows], vb)
        _add_chunk(a0, b0, _RCH)
        pltpu.sync_copy(va, out_r.at[rows])


_sc_add = pl.kernel(
    _sc_add_body,
    out_type=(
        jax.ShapeDtypeStruct((_N_ENT, _D), jnp.float32),
        jax.ShapeDtypeStruct((_N_REL, _D), jnp.float32),
    ),
    mesh=plsc.VectorSubcoreMesh(
        core_axis_name="c",
        subcore_axis_name="s",
        num_cores=_NUM_CORES,
        num_subcores=_NUM_SUBCORES,
    ),
    scratch_types=[
        pltpu.VMEM((_CH, _D), jnp.float32),
        pltpu.VMEM((_CH, _D), jnp.float32),
        pltpu.VMEM((_CH, _D), jnp.float32),
        pltpu.VMEM((_CH, _D), jnp.float32),
        pltpu.SemaphoreType.DMA,
        pltpu.SemaphoreType.DMA,
        pltpu.SemaphoreType.DMA,
        pltpu.SemaphoreType.DMA,
    ],
)


def kernel(inputs, ent_embeds_0, rel_embeds_0, ent_embeds_1, rel_embeds_1):
    del inputs
    return _sc_add(ent_embeds_0, ent_embeds_1, rel_embeds_0, rel_embeds_1)
